# baseline (speedup 1.0000x reference)
import jax
import jax.numpy as jnp
from jax.experimental import pallas as pl
from jax.experimental.pallas import tpu as pltpu

_EPS = 1e-05
_BLOCK = 16384
_CHUNK = 2048


def _align_body(g_ref, q_ref, ixoz_ref, mxoz_ref, ixoy_ref, mxoy_ref,
                iyoz_ref, myoz_ref):
    for c in range(_BLOCK // _CHUNK):
        lo = c * _CHUNK
        gt = g_ref[lo:lo + _CHUNK, :].T
        qt = q_ref[lo:lo + _CHUNK, :].T
        glo = gt[0:3, :] - 0.5 * gt[3:6, :]
        ghi = gt[0:3, :] + 0.5 * gt[3:6, :]
        qlo = qt[0:3, :] - 0.5 * qt[3:6, :]
        qhi = qt[0:3, :] + 0.5 * qt[3:6, :]
        iw = jnp.minimum(ghi, qhi) - jnp.maximum(glo, qlo) + _EPS
        mw = jnp.maximum(ghi, qhi) - jnp.minimum(glo, qlo) + _EPS

        def _plane(a, b, i_ref, m_ref):
            wa, wb = iw[a:a + 1, :], iw[b:b + 1, :]
            inter = jnp.where((wa > 0.0) & (wb > 0.0), wa * wb, 0.0)
            mbr = mw[a:a + 1, :] * mw[b:b + 1, :]
            i_ref[lo:lo + _CHUNK] = inter.reshape(_CHUNK)
            m_ref[lo:lo + _CHUNK] = mbr.reshape(_CHUNK)

        _plane(0, 2, ixoz_ref, mxoz_ref)
        _plane(0, 1, ixoy_ref, mxoy_ref)
        _plane(1, 2, iyoz_ref, myoz_ref)


def kernel(gboxes, qboxes):
    n = gboxes.shape[0]
    grid = (pl.cdiv(n, _BLOCK),)
    in_spec = pl.BlockSpec((_BLOCK, 7), lambda i: (i, 0))
    out_spec = pl.BlockSpec((_BLOCK,), lambda i: (i,))
    out_shape = tuple(jax.ShapeDtypeStruct((n,), jnp.float32)
                      for _ in range(6))
    return pl.pallas_call(
        _align_body,
        out_shape=out_shape,
        grid=grid,
        in_specs=[in_spec, in_spec],
        out_specs=[out_spec] * 6,
        compiler_params=pltpu.CompilerParams(
            dimension_semantics=("parallel",),
            vmem_limit_bytes=60 * 1024 * 1024,
        ),
        name="align_inter_aligned",
    )(gboxes, qboxes)
